# baseline (device time: 9405 ns/iter reference)
import jax
import jax.numpy as jnp
from jax import lax
from jax.experimental import pallas as pl
from jax.experimental.pallas import tpu as pltpu

N_DEV = 8
N_CHUNKS = 4
BLKS_PER_CHUNK = N_DEV // N_CHUNKS


def kernel(x, w_mat):
    m_per, k = x.shape
    _, n = w_mat.shape
    blk = n // N_DEV
    chunk = n // N_CHUNKS

    def body(x_hbm, w_hbm, out_ref, x_ref, w_ref, ysend_ref,
             load_sems, store_sem, send_sems, recv_sems):
        my = lax.axis_index("i")

        barrier_sem = pltpu.get_barrier_semaphore()
        for d in range(1, N_DEV):
            pl.semaphore_signal(
                barrier_sem, inc=1,
                device_id=((my + d) % N_DEV,),
                device_id_type=pl.DeviceIdType.MESH,
            )

        xcp = pltpu.make_async_copy(x_hbm, x_ref, load_sems.at[0])
        xcp.start()
        wcps = []
        for d in range(N_DEV):
            b = (my + 1 + d) % N_DEV
            wcp = pltpu.make_async_copy(
                w_hbm.at[:, pl.ds(b * blk, blk)],
                w_ref.at[:, pl.ds(d * blk, blk)],
                load_sems.at[d + 1],
            )
            wcp.start()
            wcps.append(wcp)
        xcp.wait()
        xb = x_ref[...].astype(jnp.bfloat16)

        sends = []
        for p in range(N_DEV // 2):
            wcps[2 * p].wait()
            wcps[2 * p + 1].wait()
            y = jnp.dot(
                xb,
                w_ref[:, 2 * p * blk:(2 * p + 2) * blk].astype(jnp.bfloat16),
                preferred_element_type=jnp.float32,
            )
            yb = jnp.maximum(y, 0.0).astype(jnp.bfloat16)
            ysend_ref[2 * p] = yb[:, :blk]
            ysend_ref[2 * p + 1] = yb[:, blk:]
            if p == 0:
                pl.semaphore_wait(barrier_sem, N_DEV - 1)
            for d in (2 * p, 2 * p + 1):
                if d < N_DEV - 1:
                    tgt = (my + 1 + d) % N_DEV
                    rdma = pltpu.make_async_remote_copy(
                        src_ref=ysend_ref.at[d],
                        dst_ref=out_ref.at[pl.ds(my * m_per, m_per), :],
                        send_sem=send_sems.at[d],
                        recv_sem=recv_sems.at[my],
                        device_id=(tgt,),
                        device_id_type=pl.DeviceIdType.MESH,
                    )
                    rdma.start()
                    sends.append(rdma)

        own = pltpu.make_async_copy(
            ysend_ref.at[N_DEV - 1],
            out_ref.at[pl.ds(my * m_per, m_per), :],
            store_sem,
        )
        own.start()

        for d in range(1, N_DEV):
            src = (my - d) % N_DEV
            recv = pltpu.make_async_remote_copy(
                src_ref=ysend_ref.at[src],
                dst_ref=out_ref.at[pl.ds(src * m_per, m_per), :],
                send_sem=send_sems.at[d - 1],
                recv_sem=recv_sems.at[src],
                device_id=(src,),
                device_id_type=pl.DeviceIdType.MESH,
            )
            recv.wait_recv()

        own.wait()
        for rdma in sends:
            rdma.wait_send()

    out_shape = jax.ShapeDtypeStruct((N_DEV * m_per, blk), jnp.bfloat16)
    return pl.pallas_call(
        body,
        out_shape=out_shape,
        in_specs=[
            pl.BlockSpec(memory_space=pl.ANY),
            pl.BlockSpec(memory_space=pl.ANY),
        ],
        out_specs=pl.BlockSpec(memory_space=pl.ANY),
        scratch_shapes=[
            pltpu.VMEM((m_per, k), jnp.float32),
            pltpu.VMEM((k, n), jnp.float32),
            pltpu.VMEM((N_DEV, m_per, blk), jnp.bfloat16),
            pltpu.SemaphoreType.DMA((N_DEV + 1,)),
            pltpu.SemaphoreType.DMA,
            pltpu.SemaphoreType.DMA((N_DEV - 1,)),
            pltpu.SemaphoreType.DMA((N_DEV,)),
        ],
        compiler_params=pltpu.CompilerParams(collective_id=0),
    )(
        pltpu.with_memory_space_constraint(x, pltpu.MemorySpace.HBM),
        pltpu.with_memory_space_constraint(w_mat, pltpu.MemorySpace.HBM),
    )


# device time: 8991 ns/iter; 1.0460x vs baseline; 1.0460x over previous
import jax
import jax.numpy as jnp
from jax import lax
from jax.experimental import pallas as pl
from jax.experimental.pallas import tpu as pltpu

N_DEV = 8
N_CHUNKS = 4
BLKS_PER_CHUNK = N_DEV // N_CHUNKS


def kernel(x, w_mat):
    m_per, k = x.shape
    _, n = w_mat.shape
    blk = n // N_DEV
    chunk = n // N_CHUNKS

    def body(x_hbm, w_hbm, out_ref, x_ref, w_ref, ysend_ref,
             load_sems, store_sem, send_sems, recv_sems):
        my = lax.axis_index("i")

        barrier_sem = pltpu.get_barrier_semaphore()
        for d in range(1, N_DEV):
            pl.semaphore_signal(
                barrier_sem, inc=1,
                device_id=((my + d) % N_DEV,),
                device_id_type=pl.DeviceIdType.MESH,
            )

        xcp = pltpu.make_async_copy(x_hbm, x_ref, load_sems.at[0])
        xcp.start()
        wcps = []
        for c in range(N_CHUNKS):
            wcp = pltpu.make_async_copy(
                w_hbm.at[:, pl.ds(c * chunk, chunk)],
                w_ref.at[:, pl.ds(c * chunk, chunk)],
                load_sems.at[c + 1],
            )
            wcp.start()
            wcps.append(wcp)
        xcp.wait()
        xb = x_ref[...].astype(jnp.bfloat16)

        sends = []

        def compute_chunk(c):
            y = jnp.dot(
                xb,
                w_ref[:, c * chunk:(c + 1) * chunk].astype(jnp.bfloat16),
                preferred_element_type=jnp.float32,
            )
            yb = jnp.maximum(y, 0.0).astype(jnp.bfloat16)
            for j in range(BLKS_PER_CHUNK):
                ysend_ref[c * BLKS_PER_CHUNK + j] = yb[:, j * blk:(j + 1) * blk]

        def send_chunk(c):
            lo, hi = c * BLKS_PER_CHUNK, (c + 1) * BLKS_PER_CHUNK
            for d in range(1, N_DEV):
                tgt = (my + d) % N_DEV
                rdma = pltpu.make_async_remote_copy(
                    src_ref=ysend_ref.at[tgt],
                    dst_ref=out_ref.at[pl.ds(my * m_per, m_per), :],
                    send_sem=send_sems.at[d - 1],
                    recv_sem=recv_sems.at[my],
                    device_id=(tgt,),
                    device_id_type=pl.DeviceIdType.MESH,
                )

                @pl.when(jnp.logical_and(tgt >= lo, tgt < hi))
                def _():
                    rdma.start()

                if c == 0:
                    sends.append(rdma)

        for c in range(N_CHUNKS):
            wcps[c].wait()
            compute_chunk(c)
            if c == 0:
                pl.semaphore_wait(barrier_sem, N_DEV - 1)
            send_chunk(c)

        own = pltpu.make_async_copy(
            ysend_ref.at[my],
            out_ref.at[pl.ds(my * m_per, m_per), :],
            store_sem,
        )
        own.start()

        for d in range(1, N_DEV):
            src = (my - d) % N_DEV
            recv = pltpu.make_async_remote_copy(
                src_ref=ysend_ref.at[src],
                dst_ref=out_ref.at[pl.ds(src * m_per, m_per), :],
                send_sem=send_sems.at[d - 1],
                recv_sem=recv_sems.at[src],
                device_id=(src,),
                device_id_type=pl.DeviceIdType.MESH,
            )
            recv.wait_recv()

        own.wait()
        for rdma in sends:
            rdma.wait_send()

    out_shape = jax.ShapeDtypeStruct((N_DEV * m_per, blk), jnp.bfloat16)
    return pl.pallas_call(
        body,
        out_shape=out_shape,
        in_specs=[
            pl.BlockSpec(memory_space=pl.ANY),
            pl.BlockSpec(memory_space=pl.ANY),
        ],
        out_specs=pl.BlockSpec(memory_space=pl.ANY),
        scratch_shapes=[
            pltpu.VMEM((m_per, k), jnp.float32),
            pltpu.VMEM((k, n), jnp.float32),
            pltpu.VMEM((N_DEV, m_per, blk), jnp.bfloat16),
            pltpu.SemaphoreType.DMA((N_CHUNKS + 1,)),
            pltpu.SemaphoreType.DMA,
            pltpu.SemaphoreType.DMA((N_DEV - 1,)),
            pltpu.SemaphoreType.DMA((N_DEV,)),
        ],
        compiler_params=pltpu.CompilerParams(collective_id=0),
    )(
        pltpu.with_memory_space_constraint(x, pltpu.MemorySpace.HBM),
        pltpu.with_memory_space_constraint(w_mat, pltpu.MemorySpace.HBM),
    )
